# Initial kernel scaffold; baseline (speedup 1.0000x reference)
import jax
import jax.numpy as jnp
from jax.experimental import pallas as pl


def kernel(x, edge_index, W1, b1, W2, b2, g2, bt2, W3, b3, W4, b4):
    raise NotImplementedError("write your pallas kernel here")


# speedup vs baseline: 3.3329x; 3.3329x over previous
import functools

import jax
import jax.numpy as jnp
from jax import lax
from jax.experimental import pallas as pl
from jax.experimental.pallas import tpu as pltpu
from jax.experimental.pallas import tpu_sc as plsc

N = 10000
E = 320000
D = 128
EPS = 1e-5

NC = 2
NS = 16
NW = NC * NS
CHUNK = 128
NCHUNK = 80
EPW = CHUNK * NCHUNK
EPAD = EPW * NW
AGG_ROWS = 10240
ZCOPIES = AGG_ROWS // NS // CHUNK
ROWS_OUT = AGG_ROWS // NS

_mesh = plsc.VectorSubcoreMesh(
    core_axis_name="c", subcore_axis_name="s", num_cores=NC, num_subcores=NS)


@functools.partial(
    pl.kernel,
    mesh=_mesh,
    out_type=jax.ShapeDtypeStruct((NC, AGG_ROWS, D), jnp.float32),
    scratch_types=[
        pltpu.VMEM((NCHUNK, CHUNK), jnp.int32),
        pltpu.VMEM((NCHUNK, CHUNK), jnp.int32),
        pltpu.VMEM((CHUNK, D), jnp.float32),
        pltpu.VMEM_SHARED((AGG_ROWS, D), jnp.float32),
        pltpu.SemaphoreType.DMA,
    ],
)
def _sc_seg_sum(x_hbm, src_hbm, dst_hbm, out_hbm, sidx, didx, rows, agg, sem):
    c = lax.axis_index("c")
    s = lax.axis_index("s")
    wid = s * NC + c

    pltpu.sync_copy(src_hbm.at[wid], sidx)
    pltpu.sync_copy(dst_hbm.at[wid], didx)

    zeros = jnp.zeros((16,), jnp.float32)

    def _zrow(i, carry):
        for k in range(D // 16):
            rows[i, pl.ds(k * 16, 16)] = zeros
        return carry

    lax.fori_loop(0, CHUNK, _zrow, 0)
    zbase = s * (AGG_ROWS // NS)
    for t in range(ZCOPIES):
        pltpu.sync_copy(rows, agg.at[pl.ds(zbase + t * CHUNK, CHUNK)])
    plsc.subcore_barrier()

    def _chunk(j, carry):
        pltpu.async_copy(x_hbm.at[sidx.at[j]], rows, sem).wait()
        pltpu.sync_copy(rows, agg.at[didx.at[j]], add=True)
        return carry

    lax.fori_loop(0, NCHUNK, _chunk, 0)
    plsc.subcore_barrier()

    obase = s * ROWS_OUT
    pltpu.sync_copy(agg.at[pl.ds(obase, ROWS_OUT)],
                    out_hbm.at[c, pl.ds(obase, ROWS_OUT)])


BLK = 1000
GRID = N // BLK


def _mlp_stats_body(x_ref, p_ref, w1_ref, b1_ref, w2_ref, b2_ref,
                    out_ref, sum_ref, sq_ref):
    t = x_ref[...] + p_ref[0] + p_ref[1]
    u = jnp.maximum(
        jnp.dot(t, w1_ref[...], preferred_element_type=jnp.float32)
        + b1_ref[...], 0.0)
    v = (jnp.dot(u, w2_ref[...], preferred_element_type=jnp.float32)
         + b2_ref[...])
    r = jnp.maximum(v, 0.0)
    out_ref[...] = r

    @pl.when(pl.program_id(0) == 0)
    def _():
        sum_ref[...] = jnp.zeros_like(sum_ref)
        sq_ref[...] = jnp.zeros_like(sq_ref)

    sum_ref[...] += jnp.sum(r, axis=0, keepdims=True)
    sq_ref[...] += jnp.sum(r * r, axis=0, keepdims=True)


def _mlp_final_body(x_ref, p_ref, w1_ref, b1_ref, w2_ref, b2_ref, out_ref):
    t = x_ref[...] + p_ref[0] + p_ref[1]
    u = jnp.maximum(
        jnp.dot(t, w1_ref[...], preferred_element_type=jnp.float32)
        + b1_ref[...], 0.0)
    out_ref[...] = (
        jnp.dot(u, w2_ref[...], preferred_element_type=jnp.float32)
        + b2_ref[...])


def _bn_body(r_ref, sum_ref, sq_ref, g_ref, b_ref, out_ref):
    mean = sum_ref[...] * (1.0 / N)
    var = sq_ref[...] * (1.0 / N) - mean * mean
    scale = g_ref[...] * lax.rsqrt(var + EPS)
    shift = b_ref[...] - mean * scale
    out_ref[...] = r_ref[...] * scale + shift


_row_spec = pl.BlockSpec((BLK, D), lambda i: (i, 0))
_p_spec = pl.BlockSpec((NC, BLK, D), lambda i: (0, i, 0))
_w_spec = pl.BlockSpec((D, D), lambda i: (0, 0))
_vec_spec = pl.BlockSpec((1, D), lambda i: (0, 0))

_mlp_stats = pl.pallas_call(
    _mlp_stats_body,
    grid=(GRID,),
    in_specs=[_row_spec, _p_spec, _w_spec, _vec_spec, _w_spec, _vec_spec],
    out_specs=[_row_spec, _vec_spec, _vec_spec],
    out_shape=[
        jax.ShapeDtypeStruct((N, D), jnp.float32),
        jax.ShapeDtypeStruct((1, D), jnp.float32),
        jax.ShapeDtypeStruct((1, D), jnp.float32),
    ],
)

_mlp_final = pl.pallas_call(
    _mlp_final_body,
    grid=(GRID,),
    in_specs=[_row_spec, _p_spec, _w_spec, _vec_spec, _w_spec, _vec_spec],
    out_specs=_row_spec,
    out_shape=jax.ShapeDtypeStruct((N, D), jnp.float32),
)

_bn = pl.pallas_call(
    _bn_body,
    grid=(GRID,),
    in_specs=[_row_spec, _vec_spec, _vec_spec, _vec_spec, _vec_spec],
    out_specs=_row_spec,
    out_shape=jax.ShapeDtypeStruct((N, D), jnp.float32),
)


def kernel(x, edge_index, W1, b1, W2, b2, g2, bt2, W3, b3, W4, b4):
    src = edge_index[0].astype(jnp.int32)
    dst = edge_index[1].astype(jnp.int32)
    npad = EPAD - E
    src = jnp.concatenate([src, jnp.zeros((npad,), jnp.int32)])
    dst = jnp.concatenate([dst, jnp.full((npad,), N, jnp.int32)])
    src = src.reshape(NW, NCHUNK, CHUNK)
    dst = dst.reshape(NW, NCHUNK, CHUNK)

    b1r = b1.reshape(1, D)
    b2r = b2.reshape(1, D)
    b3r = b3.reshape(1, D)
    b4r = b4.reshape(1, D)

    p = _sc_seg_sum(x, src, dst)
    r, csum, csq = _mlp_stats(x, p, W1, b1r, W2, b2r)
    h = _bn(r, csum, csq, g2.reshape(1, D), bt2.reshape(1, D))
    q = _sc_seg_sum(h, src, dst)
    return _mlp_final(h, q, W3, b3r, W4, b4r)


# speedup vs baseline: 3.7825x; 1.1349x over previous
import functools

import jax
import jax.numpy as jnp
from jax import lax
from jax.experimental import pallas as pl
from jax.experimental.pallas import tpu as pltpu
from jax.experimental.pallas import tpu_sc as plsc

N = 10000
E = 320000
D = 128
EPS = 1e-5

NC = 2
NS = 16
NW = NC * NS
CHUNK = 128
NCHUNK = 80
EPW = CHUNK * NCHUNK
EPAD = EPW * NW
AGG_ROWS = 10240
ZCOPIES = AGG_ROWS // NS // CHUNK
ROWS_OUT = AGG_ROWS // NS
NBUF = 2
IDX_BITS = 14
IDX_MASK = (1 << IDX_BITS) - 1

_mesh = plsc.VectorSubcoreMesh(
    core_axis_name="c", subcore_axis_name="s", num_cores=NC, num_subcores=NS)


@functools.partial(
    pl.kernel,
    mesh=_mesh,
    out_type=jax.ShapeDtypeStruct((NC, AGG_ROWS, D), jnp.float32),
    scratch_types=[
        pltpu.VMEM((NCHUNK, CHUNK), jnp.int32),
        pltpu.VMEM((NBUF, 2, CHUNK), jnp.int32),
        pltpu.VMEM((NBUF, CHUNK, D), jnp.float32),
        pltpu.VMEM_SHARED((AGG_ROWS, D), jnp.float32),
        pltpu.SemaphoreType.DMA,
    ],
)
def _sc_seg_sum(x_hbm, pidx_hbm, out_hbm, pidx, unpk, rows, agg, sem):
    c = lax.axis_index("c")
    s = lax.axis_index("s")
    wid = s * NC + c

    pltpu.sync_copy(pidx_hbm.at[wid], pidx)

    zeros = jnp.zeros((16,), jnp.float32)

    def _zrow(i, carry):
        for k in range(D // 16):
            rows[0, i, pl.ds(k * 16, 16)] = zeros
        return carry

    lax.fori_loop(0, CHUNK, _zrow, 0)
    zbase = s * (AGG_ROWS // NS)
    for t in range(ZCOPIES):
        pltpu.sync_copy(rows.at[0], agg.at[pl.ds(zbase + t * CHUNK, CHUNK)])
    plsc.subcore_barrier()

    def _unpack(j, b):
        for k in range(CHUNK // 16):
            w = pidx[j, pl.ds(k * 16, 16)]
            unpk[b, 0, pl.ds(k * 16, 16)] = lax.bitwise_and(w, IDX_MASK)
            unpk[b, 1, pl.ds(k * 16, 16)] = lax.shift_right_logical(w, IDX_BITS)

    for b in range(NBUF):
        _unpack(b, b)
        pltpu.async_copy(x_hbm.at[unpk.at[b, 0]], rows.at[b], sem)

    def _group(g, carry):
        for b in range(NBUF):
            j = g * NBUF + b
            pltpu.make_async_copy(x_hbm.at[unpk.at[b, 0]], rows.at[b], sem).wait()
            pltpu.sync_copy(rows.at[b], agg.at[unpk.at[b, 1]], add=True)
            _unpack(j + NBUF, b)
            pltpu.async_copy(x_hbm.at[unpk.at[b, 0]], rows.at[b], sem)
        return carry

    lax.fori_loop(0, NCHUNK // NBUF - 1, _group, 0)
    for b in range(NBUF):
        pltpu.make_async_copy(x_hbm.at[unpk.at[b, 0]], rows.at[b], sem).wait()
        pltpu.sync_copy(rows.at[b], agg.at[unpk.at[b, 1]], add=True)
    plsc.subcore_barrier()

    obase = s * ROWS_OUT
    pltpu.sync_copy(agg.at[pl.ds(obase, ROWS_OUT)],
                    out_hbm.at[c, pl.ds(obase, ROWS_OUT)])


BLK = 1000
GRID = N // BLK


def _mlp_stats_body(x_ref, p_ref, w1_ref, b1_ref, w2_ref, b2_ref,
                    out_ref, sum_ref, sq_ref):
    t = x_ref[...] + p_ref[0] + p_ref[1]
    u = jnp.maximum(
        jnp.dot(t, w1_ref[...], preferred_element_type=jnp.float32)
        + b1_ref[...], 0.0)
    v = (jnp.dot(u, w2_ref[...], preferred_element_type=jnp.float32)
         + b2_ref[...])
    r = jnp.maximum(v, 0.0)
    out_ref[...] = r

    @pl.when(pl.program_id(0) == 0)
    def _():
        sum_ref[...] = jnp.zeros_like(sum_ref)
        sq_ref[...] = jnp.zeros_like(sq_ref)

    sum_ref[...] += jnp.sum(r, axis=0, keepdims=True)
    sq_ref[...] += jnp.sum(r * r, axis=0, keepdims=True)


def _mlp_final_body(x_ref, p_ref, w1_ref, b1_ref, w2_ref, b2_ref, out_ref):
    t = x_ref[...] + p_ref[0] + p_ref[1]
    u = jnp.maximum(
        jnp.dot(t, w1_ref[...], preferred_element_type=jnp.float32)
        + b1_ref[...], 0.0)
    out_ref[...] = (
        jnp.dot(u, w2_ref[...], preferred_element_type=jnp.float32)
        + b2_ref[...])


def _bn_body(r_ref, sum_ref, sq_ref, g_ref, b_ref, out_ref):
    mean = sum_ref[...] * (1.0 / N)
    var = sq_ref[...] * (1.0 / N) - mean * mean
    scale = g_ref[...] * lax.rsqrt(var + EPS)
    shift = b_ref[...] - mean * scale
    out_ref[...] = r_ref[...] * scale + shift


_row_spec = pl.BlockSpec((BLK, D), lambda i: (i, 0))
_p_spec = pl.BlockSpec((NC, BLK, D), lambda i: (0, i, 0))
_w_spec = pl.BlockSpec((D, D), lambda i: (0, 0))
_vec_spec = pl.BlockSpec((1, D), lambda i: (0, 0))

_mlp_stats = pl.pallas_call(
    _mlp_stats_body,
    grid=(GRID,),
    in_specs=[_row_spec, _p_spec, _w_spec, _vec_spec, _w_spec, _vec_spec],
    out_specs=[_row_spec, _vec_spec, _vec_spec],
    out_shape=[
        jax.ShapeDtypeStruct((N, D), jnp.float32),
        jax.ShapeDtypeStruct((1, D), jnp.float32),
        jax.ShapeDtypeStruct((1, D), jnp.float32),
    ],
)

_mlp_final = pl.pallas_call(
    _mlp_final_body,
    grid=(GRID,),
    in_specs=[_row_spec, _p_spec, _w_spec, _vec_spec, _w_spec, _vec_spec],
    out_specs=_row_spec,
    out_shape=jax.ShapeDtypeStruct((N, D), jnp.float32),
)

_bn = pl.pallas_call(
    _bn_body,
    grid=(GRID,),
    in_specs=[_row_spec, _vec_spec, _vec_spec, _vec_spec, _vec_spec],
    out_specs=_row_spec,
    out_shape=jax.ShapeDtypeStruct((N, D), jnp.float32),
)


def kernel(x, edge_index, W1, b1, W2, b2, g2, bt2, W3, b3, W4, b4):
    src = edge_index[0].astype(jnp.int32)
    dst = edge_index[1].astype(jnp.int32)
    packed = src | (dst << IDX_BITS)
    npad = EPAD - E
    packed = jnp.concatenate(
        [packed, jnp.full((npad,), N << IDX_BITS, jnp.int32)])
    pidx = packed.reshape(NW, NCHUNK, CHUNK)

    b1r = b1.reshape(1, D)
    b2r = b2.reshape(1, D)
    b3r = b3.reshape(1, D)
    b4r = b4.reshape(1, D)

    p = _sc_seg_sum(x, pidx)
    r, csum, csq = _mlp_stats(x, p, W1, b1r, W2, b2r)
    h = _bn(r, csum, csq, g2.reshape(1, D), bt2.reshape(1, D))
    q = _sc_seg_sum(h, pidx)
    return _mlp_final(h, q, W3, b3r, W4, b4r)
